# baseline (speedup 1.0000x reference)
import functools

import jax
import jax.numpy as jnp
from jax import lax
from jax.experimental import pallas as pl
from jax.experimental.pallas import tpu as pltpu
from jax.experimental.pallas import tpu_sc as plsc

VOCAB = 1000
D_MODEL = 64
B_TOK = 1024
T_SEQ = 50
T_CHUNK = 25

NUM_CORES = 2
NUM_SUBCORES = 16
NUM_WORKERS = NUM_CORES * NUM_SUBCORES

NTB_C = T_CHUNK * 8
FULL_UNITS = NTB_C // NUM_WORKERS
TAIL_UNITS = NTB_C - FULL_UNITS * NUM_WORKERS


def _transpose_unit(wbuf, tbuf):
    iota = lax.iota(jnp.int32, 16)
    rowidx = [iota + 16 * c for c in range(8)]

    def dbody(dt, carry):
        base = dt * 8
        for ds in range(8):
            col = jnp.full((16,), base + ds, jnp.int32)
            vecs = [plsc.load_gather(wbuf, [rowidx[c], col])
                    for c in range(8)]
            for c in range(8):
                tbuf[dt, ds, pl.ds(c * 16, 16)] = vecs[c]
        return carry

    lax.fori_loop(0, 8, dbody, 0)


def _make_gather(chunk):
    unit_offset = chunk * NTB_C

    @functools.partial(
        pl.kernel,
        out_type=jax.ShapeDtypeStruct((T_CHUNK, 8, 8, 8, 128), jnp.float32),
        mesh=plsc.VectorSubcoreMesh(core_axis_name="c", subcore_axis_name="s"),
        scratch_types=[
            pltpu.VMEM(((FULL_UNITS + 1) * 128,), jnp.int32),
            pltpu.VMEM((FULL_UNITS + 1, 128, D_MODEL), jnp.float32),
            pltpu.VMEM((2, 8, 8, 128), jnp.float32),
            [pltpu.SemaphoreType.DMA] * (FULL_UNITS + 1),
            [pltpu.SemaphoreType.DMA] * 2,
        ],
        compiler_params=pltpu.CompilerParams(
            use_tc_tiling_on_sc=False, needs_layout_passes=False),
        name=f"embed_gather_c{chunk}",
    )
    def _gather_kernel(idt_hbm, emb_hbm, out_hbm,
                       idx_v, wbuf, tbuf, gsems, wsems):
        wid = lax.axis_index("s") * NUM_CORES + lax.axis_index("c")
        base_tb = wid * FULL_UNITS
        pltpu.sync_copy(
            idt_hbm.at[pl.ds((unit_offset + base_tb) * 128,
                             FULL_UNITS * 128)],
            idx_v.at[pl.ds(0, FULL_UNITS * 128)])

        def idx_slice(j):
            return idx_v.at[pl.ds(j * 128, 128)]

        def dst(tb):
            return out_hbm.at[tb // 8, :, tb % 8]

        is_tail = wid < TAIL_UNITS
        tail_tb = NUM_WORKERS * FULL_UNITS + wid

        @pl.when(is_tail)
        def _():
            pltpu.sync_copy(
                idt_hbm.at[pl.ds((unit_offset + tail_tb) * 128, 128)],
                idx_v.at[pl.ds(FULL_UNITS * 128, 128)])

        gd = [pltpu.async_copy(emb_hbm.at[idx_slice(j)], wbuf.at[j],
                               gsems[j])
              for j in range(FULL_UNITS)]

        @pl.when(is_tail)
        def _():
            pltpu.async_copy(emb_hbm.at[idx_slice(FULL_UNITS)],
                             wbuf.at[FULL_UNITS], gsems[FULL_UNITS])

        wd = [None] * FULL_UNITS
        for j in range(FULL_UNITS):
            p = j & 1
            gd[j].wait()
            if j >= 2:
                wd[j - 2].wait()
            _transpose_unit(wbuf.at[j], tbuf.at[p])
            wd[j] = pltpu.async_copy(tbuf.at[p], dst(base_tb + j),
                                     wsems[p])
        wd[FULL_UNITS - 2].wait()
        wd[FULL_UNITS - 1].wait()

        @pl.when(is_tail)
        def _():
            pltpu.make_async_copy(emb_hbm.at[idx_slice(FULL_UNITS)],
                                  wbuf.at[FULL_UNITS],
                                  gsems[FULL_UNITS]).wait()
            _transpose_unit(wbuf.at[FULL_UNITS], tbuf.at[0])
            pltpu.sync_copy(tbuf.at[0], dst(tail_tb))

    return _gather_kernel


_GATHERS = [_make_gather(0), _make_gather(1)]


def _matmul_body(x_ref, p_ref, o_ref):
    o_ref[0] = lax.dot_general(
        p_ref[...], x_ref[0],
        dimension_numbers=(((1,), (0,)), ((), ())),
        preferred_element_type=jnp.float32,
    )


def _matmul_body2(x_ref, p_ref, big_ref, o_ref):
    o_ref[0] = lax.dot_general(
        p_ref[...], x_ref[0],
        dimension_numbers=(((1,), (0,)), ((), ())),
        preferred_element_type=jnp.float32,
    )


def _project_chunk1(xt, proj_W):
    return pl.pallas_call(
        _matmul_body,
        grid=(T_CHUNK,),
        in_specs=[
            pl.BlockSpec((1, D_MODEL, B_TOK), lambda t: (t, 0, 0)),
            pl.BlockSpec((VOCAB, D_MODEL), lambda t: (0, 0)),
        ],
        out_specs=pl.BlockSpec((1, VOCAB, B_TOK), lambda t: (t, 0, 0)),
        out_shape=jax.ShapeDtypeStruct((T_SEQ, VOCAB, B_TOK), jnp.float32),
    )(xt, proj_W)


def _project_chunk2(xt, proj_W, big):
    return pl.pallas_call(
        _matmul_body2,
        grid=(T_CHUNK,),
        in_specs=[
            pl.BlockSpec((1, D_MODEL, B_TOK), lambda t: (t, 0, 0)),
            pl.BlockSpec((VOCAB, D_MODEL), lambda t: (0, 0)),
            pl.BlockSpec((1, 8, 128), lambda t: (0, 0, 0)),
        ],
        out_specs=pl.BlockSpec((1, VOCAB, B_TOK),
                               lambda t: (t + T_CHUNK, 0, 0)),
        out_shape=jax.ShapeDtypeStruct((T_SEQ, VOCAB, B_TOK), jnp.float32),
        input_output_aliases={2: 0},
    )(xt, proj_W, big)


def kernel(input_ids, embed_table, proj_W):
    b, t = input_ids.shape
    ids_t = input_ids.T.reshape(-1).astype(jnp.int32)
    x5_0 = _GATHERS[0](ids_t, embed_table)
    x5_1 = _GATHERS[1](ids_t, embed_table)
    xt0 = x5_0.transpose(0, 1, 3, 2, 4).reshape(T_CHUNK, D_MODEL, B_TOK)
    xt1 = x5_1.transpose(0, 1, 3, 2, 4).reshape(T_CHUNK, D_MODEL, B_TOK)
    big = _project_chunk1(xt0, proj_W)
    out3 = _project_chunk2(xt1, proj_W, big)
    return out3.transpose(2, 0, 1).reshape(b, t, VOCAB)


# speedup vs baseline: 1.0297x; 1.0297x over previous
import functools

import jax
import jax.numpy as jnp
from jax import lax
from jax.experimental import pallas as pl
from jax.experimental.pallas import tpu as pltpu
from jax.experimental.pallas import tpu_sc as plsc

VOCAB = 1000
D_MODEL = 64
B_TOK = 1024
T_SEQ = 50
T_CHUNK = 25

NUM_CORES = 2
NUM_SUBCORES = 16
NUM_WORKERS = NUM_CORES * NUM_SUBCORES

NTB_C = T_CHUNK * 8
FULL_UNITS = NTB_C // NUM_WORKERS
TAIL_UNITS = NTB_C - FULL_UNITS * NUM_WORKERS


def _transpose_unit(wbuf, tbuf):
    iota = lax.iota(jnp.int32, 16)
    rowidx = [iota + 16 * c for c in range(8)]

    def dbody(dt, carry):
        base = dt * 8
        for ds in range(8):
            col = jnp.full((16,), base + ds, jnp.int32)
            vecs = [plsc.load_gather(wbuf, [rowidx[c], col])
                    for c in range(8)]
            for c in range(8):
                tbuf[dt, ds, pl.ds(c * 16, 16)] = vecs[c]
        return carry

    lax.fori_loop(0, 8, dbody, 0)


def _make_gather(chunk):
    unit_offset = chunk * NTB_C

    @functools.partial(
        pl.kernel,
        out_type=jax.ShapeDtypeStruct((T_CHUNK, 8, 8, 8, 128), jnp.float32),
        mesh=plsc.VectorSubcoreMesh(core_axis_name="c", subcore_axis_name="s"),
        scratch_types=[
            pltpu.VMEM(((FULL_UNITS + 1) * 128,), jnp.int32),
            pltpu.VMEM((2, 128, D_MODEL), jnp.float32),
            pltpu.VMEM((2, 8, 8, 128), jnp.float32),
            [pltpu.SemaphoreType.DMA] * 2,
            [pltpu.SemaphoreType.DMA] * 2,
        ],
        compiler_params=pltpu.CompilerParams(
            use_tc_tiling_on_sc=False, needs_layout_passes=False),
        name=f"embed_gather_c{chunk}",
    )
    def _gather_kernel(idt_hbm, emb_hbm, out_hbm,
                       idx_v, wbuf, tbuf, gsems, wsems):
        wid = lax.axis_index("s") * NUM_CORES + lax.axis_index("c")
        base_tb = wid * FULL_UNITS
        pltpu.sync_copy(
            idt_hbm.at[pl.ds((unit_offset + base_tb) * 128,
                             FULL_UNITS * 128)],
            idx_v.at[pl.ds(0, FULL_UNITS * 128)])

        def idx_slice(j):
            return idx_v.at[pl.ds(j * 128, 128)]

        def dst(tb):
            return out_hbm.at[tb // 8, :, tb % 8]

        gd = [None] * FULL_UNITS
        wd = [None] * FULL_UNITS
        gd[0] = pltpu.async_copy(emb_hbm.at[idx_slice(0)], wbuf.at[0],
                                 gsems[0])
        for j in range(FULL_UNITS):
            p = j & 1
            if j + 1 < FULL_UNITS:
                gd[j + 1] = pltpu.async_copy(
                    emb_hbm.at[idx_slice(j + 1)], wbuf.at[1 - p],
                    gsems[1 - p])
            gd[j].wait()
            if j >= 2:
                wd[j - 2].wait()
            _transpose_unit(wbuf.at[p], tbuf.at[p])
            wd[j] = pltpu.async_copy(tbuf.at[p], dst(base_tb + j),
                                     wsems[p])
        wd[FULL_UNITS - 2].wait()
        wd[FULL_UNITS - 1].wait()

        @pl.when(wid < TAIL_UNITS)
        def _():
            tb = NUM_WORKERS * FULL_UNITS + wid
            pltpu.sync_copy(
                idt_hbm.at[pl.ds((unit_offset + tb) * 128, 128)],
                idx_v.at[pl.ds(FULL_UNITS * 128, 128)])
            g = pltpu.async_copy(
                emb_hbm.at[idx_v.at[pl.ds(FULL_UNITS * 128, 128)]],
                wbuf.at[0], gsems[0])
            g.wait()
            _transpose_unit(wbuf.at[0], tbuf.at[0])
            pltpu.sync_copy(tbuf.at[0], dst(tb))

    return _gather_kernel


_GATHERS = [_make_gather(0), _make_gather(1)]


def _matmul_body(x_ref, p_ref, o_ref):
    o_ref[0] = lax.dot_general(
        p_ref[...], x_ref[0],
        dimension_numbers=(((1,), (0,)), ((), ())),
        preferred_element_type=jnp.float32,
    )


def _matmul_body2(x_ref, p_ref, big_ref, o_ref):
    o_ref[0] = lax.dot_general(
        p_ref[...], x_ref[0],
        dimension_numbers=(((1,), (0,)), ((), ())),
        preferred_element_type=jnp.float32,
    )


def _project_chunk1(xt, proj_W):
    return pl.pallas_call(
        _matmul_body,
        grid=(T_CHUNK,),
        in_specs=[
            pl.BlockSpec((1, D_MODEL, B_TOK), lambda t: (t, 0, 0)),
            pl.BlockSpec((VOCAB, D_MODEL), lambda t: (0, 0)),
        ],
        out_specs=pl.BlockSpec((1, VOCAB, B_TOK), lambda t: (t, 0, 0)),
        out_shape=jax.ShapeDtypeStruct((T_SEQ, VOCAB, B_TOK), jnp.float32),
    )(xt, proj_W)


def _project_chunk2(xt, proj_W, big):
    return pl.pallas_call(
        _matmul_body2,
        grid=(T_CHUNK,),
        in_specs=[
            pl.BlockSpec((1, D_MODEL, B_TOK), lambda t: (t, 0, 0)),
            pl.BlockSpec((VOCAB, D_MODEL), lambda t: (0, 0)),
            pl.BlockSpec((1, 8, 128), lambda t: (0, 0, 0)),
        ],
        out_specs=pl.BlockSpec((1, VOCAB, B_TOK),
                               lambda t: (t + T_CHUNK, 0, 0)),
        out_shape=jax.ShapeDtypeStruct((T_SEQ, VOCAB, B_TOK), jnp.float32),
        input_output_aliases={2: 0},
    )(xt, proj_W, big)


def kernel(input_ids, embed_table, proj_W):
    b, t = input_ids.shape
    ids_t = input_ids.T.reshape(-1).astype(jnp.int32)
    x5_0 = _GATHERS[0](ids_t, embed_table)
    x5_1 = _GATHERS[1](ids_t, embed_table)
    xt0 = x5_0.transpose(0, 1, 3, 2, 4).reshape(T_CHUNK, D_MODEL, B_TOK)
    xt1 = x5_1.transpose(0, 1, 3, 2, 4).reshape(T_CHUNK, D_MODEL, B_TOK)
    big = _project_chunk1(xt0, proj_W)
    out3 = _project_chunk2(xt1, proj_W, big)
    return out3.transpose(2, 0, 1).reshape(b, t, VOCAB)
